# Initial kernel scaffold; baseline (speedup 1.0000x reference)
import jax
import jax.numpy as jnp
from jax.experimental import pallas as pl


def kernel(node_features, W):
    raise NotImplementedError("write your pallas kernel here")


# speedup vs baseline: 5.4020x; 5.4020x over previous
import functools

import jax
import jax.numpy as jnp
from jax.experimental import pallas as pl
from jax.experimental.pallas import tpu as pltpu

_N = 2048
_D = 256
_P = 16
_EPS = 0.1
_K = 30

_BN = 256


def _build_z_kernel(x_ref, w_ref, z_ref):
    x = x_ref[...]
    w = w_ref[...]
    y = x[:, None, :] * w[None, :, :]
    ss = jnp.sum(y * y, axis=-1, keepdims=True)
    norm = jnp.maximum(jnp.sqrt(ss), 1e-12)
    z = (y / norm) * 0.25
    z_ref[...] = z.reshape(x.shape[0], _P * _D)


def _adj_topk_kernel(a_ref, b_ref, out_ref):
    a = a_ref[...]
    b = b_ref[...]
    adj = jax.lax.dot_general(
        a, b, (((1,), (1,)), ((), ())),
        preferred_element_type=jnp.float32)
    masked = jnp.where(adj > _EPS, adj, 0.0)

    def body(_, carry):
        work, _m = carry
        m = jnp.max(work, axis=1, keepdims=True)
        work = jnp.where(work == m, 0.0, work)
        return work, m

    _, thresh = jax.lax.fori_loop(
        0, _K, body, (masked, jnp.zeros((a.shape[0], 1), jnp.float32)))
    out_ref[...] = jnp.where((masked >= thresh) & (masked > 0.0), masked, 0.0)


@jax.jit
def kernel(node_features, W):
    n, d = node_features.shape
    p = W.shape[0]
    pd = p * d

    z = pl.pallas_call(
        _build_z_kernel,
        grid=(n // _BN,),
        in_specs=[
            pl.BlockSpec((_BN, d), lambda i: (i, 0)),
            pl.BlockSpec((p, d), lambda i: (0, 0)),
        ],
        out_specs=pl.BlockSpec((_BN, pd), lambda i: (i, 0)),
        out_shape=jax.ShapeDtypeStruct((n, pd), jnp.float32),
    )(node_features, W)

    out = pl.pallas_call(
        _adj_topk_kernel,
        grid=(n // _BN,),
        in_specs=[
            pl.BlockSpec((_BN, pd), lambda i: (i, 0)),
            pl.BlockSpec((n, pd), lambda i: (0, 0)),
        ],
        out_specs=pl.BlockSpec((_BN, n), lambda i: (i, 0)),
        out_shape=jax.ShapeDtypeStruct((n, n), jnp.float32),
    )(z, z)
    return out
